# Initial kernel scaffold; baseline (speedup 1.0000x reference)
import jax
import jax.numpy as jnp
from jax.experimental import pallas as pl


def kernel(inputs, embeddings):
    raise NotImplementedError("write your pallas kernel here")


# speedup vs baseline: 1.3867x; 1.3867x over previous
import functools

import jax
import jax.numpy as jnp
from jax import lax
from jax.experimental import pallas as pl
from jax.experimental.pallas import tpu as pltpu
from jax.experimental.pallas import tpu_sc as plsc

_BT = 256
_K = 8192
_D = 256


def _argmin_body(x_ref, e_ref, codes_ref):
    x = x_ref[...]
    e = e_ref[...]
    dot = lax.dot_general(x, e, (((1,), (1,)), ((), ())),
                          preferred_element_type=jnp.float32)
    l2x = jnp.sum(x * x, axis=1, keepdims=True)
    l2e = jnp.sum(e * e, axis=1)[None, :]
    dist = l2x + l2e - 2.0 * dot
    m = jnp.min(dist, axis=1, keepdims=True)
    iota = lax.broadcasted_iota(jnp.int32, dist.shape, 1)
    codes = jnp.min(jnp.where(dist == m, iota, _K), axis=1)
    codes_ref[0, 0, :] = codes


def _codes_pallas(x, embeddings):
    n = x.shape[0]
    nb = n // _BT
    codes3 = pl.pallas_call(
        _argmin_body,
        grid=(nb,),
        in_specs=[
            pl.BlockSpec((_BT, _D), lambda i: (i, 0)),
            pl.BlockSpec((_K, _D), lambda i: (0, 0)),
        ],
        out_specs=pl.BlockSpec((1, 1, _BT), lambda i: (i, 0, 0)),
        out_shape=jax.ShapeDtypeStruct((nb, 1, _BT), jnp.int32),
    )(x, embeddings)
    return codes3.reshape(n)


def _gather_sc(embeddings, codes_flat):
    info = plsc.get_sparse_core_info()
    nw = info.num_cores * info.num_subcores
    b = codes_flat.shape[0]
    b_per_w = b // nw
    ch = 256
    nch = b_per_w // ch
    mesh = plsc.VectorSubcoreMesh(core_axis_name="c", subcore_axis_name="s")

    @functools.partial(
        pl.kernel, mesh=mesh,
        out_type=jax.ShapeDtypeStruct((b, _D), jnp.float32),
        scratch_types=[
            pltpu.VMEM((b_per_w,), jnp.int32),
            pltpu.VMEM((ch, _D), jnp.float32),
            pltpu.SemaphoreType.DMA,
        ],
    )
    def k(table_hbm, idx_hbm, out_hbm, idx_v, rows_v, sem):
        wid = lax.axis_index("s") * info.num_cores + lax.axis_index("c")
        base = wid * b_per_w
        pltpu.sync_copy(idx_hbm.at[pl.ds(base, b_per_w)], idx_v)
        for c in range(nch):
            pltpu.async_copy(
                table_hbm.at[idx_v.at[pl.ds(c * ch, ch)]], rows_v, sem
            ).wait()
            pltpu.sync_copy(rows_v, out_hbm.at[pl.ds(base + c * ch, ch)])

    return k(embeddings, codes_flat)


def kernel(inputs, embeddings):
    bsz, h, w, d = inputs.shape
    n = bsz * h * w
    x = inputs.reshape(n, d)
    codes_flat = _codes_pallas(x, embeddings)
    code_vecs = _gather_sc(embeddings, codes_flat)
    return codes_flat.reshape(bsz, h, w), code_vecs.reshape(bsz, h, w, d)


# speedup vs baseline: 1.9048x; 1.3736x over previous
import functools

import jax
import jax.numpy as jnp
from jax import lax
from jax.experimental import pallas as pl
from jax.experimental.pallas import tpu as pltpu
from jax.experimental.pallas import tpu_sc as plsc

_BT = 256
_K = 8192
_D = 256


def _argmin_body(x_ref, e_ref, codes_ref, l2e_ref):
    @pl.when(pl.program_id(0) == 0)
    def _():
        e0 = e_ref[...]
        l2e_ref[...] = jnp.sum(e0 * e0, axis=1)[None, :]

    x = x_ref[...]
    e = e_ref[...]
    dot = lax.dot_general(x, e, (((1,), (1,)), ((), ())),
                          preferred_element_type=jnp.float32)
    l2x = jnp.sum(x * x, axis=1, keepdims=True)
    l2e = l2e_ref[...]
    dist = l2x + l2e - 2.0 * dot
    m = jnp.min(dist, axis=1, keepdims=True)
    iota = lax.broadcasted_iota(jnp.int32, dist.shape, 1).astype(jnp.float32)
    codes_f = jnp.min(jnp.where(dist == m, iota, jnp.float32(_K)), axis=1)
    codes_ref[0, 0, :] = codes_f.astype(jnp.int32)


def _codes_pallas(x, embeddings):
    n = x.shape[0]
    nb = n // _BT
    codes3 = pl.pallas_call(
        _argmin_body,
        grid=(nb,),
        in_specs=[
            pl.BlockSpec((_BT, _D), lambda i: (i, 0)),
            pl.BlockSpec((_K, _D), lambda i: (0, 0)),
        ],
        out_specs=pl.BlockSpec((1, 1, _BT), lambda i: (i, 0, 0)),
        out_shape=jax.ShapeDtypeStruct((nb, 1, _BT), jnp.int32),
        scratch_shapes=[pltpu.VMEM((1, _K), jnp.float32)],
    )(x, embeddings)
    return codes3.reshape(n)


def _gather_sc(embeddings, codes_flat):
    info = plsc.get_sparse_core_info()
    nw = info.num_cores * info.num_subcores
    b = codes_flat.shape[0]
    b_per_w = b // nw
    ch = 256
    nch = b_per_w // ch
    mesh = plsc.VectorSubcoreMesh(core_axis_name="c", subcore_axis_name="s")

    @functools.partial(
        pl.kernel, mesh=mesh,
        out_type=jax.ShapeDtypeStruct((b, _D), jnp.float32),
        scratch_types=[
            pltpu.VMEM((b_per_w,), jnp.int32),
            pltpu.VMEM((ch, _D), jnp.float32),
            pltpu.SemaphoreType.DMA,
        ],
    )
    def k(table_hbm, idx_hbm, out_hbm, idx_v, rows_v, sem):
        wid = lax.axis_index("s") * info.num_cores + lax.axis_index("c")
        base = wid * b_per_w
        pltpu.sync_copy(idx_hbm.at[pl.ds(base, b_per_w)], idx_v)
        for c in range(nch):
            pltpu.async_copy(
                table_hbm.at[idx_v.at[pl.ds(c * ch, ch)]], rows_v, sem
            ).wait()
            pltpu.sync_copy(rows_v, out_hbm.at[pl.ds(base + c * ch, ch)])

    return k(embeddings, codes_flat)


def kernel(inputs, embeddings):
    bsz, h, w, d = inputs.shape
    n = bsz * h * w
    x = inputs.reshape(n, d)
    codes_flat = _codes_pallas(x, embeddings)
    code_vecs = _gather_sc(embeddings, codes_flat)
    return codes_flat.reshape(bsz, h, w), code_vecs.reshape(bsz, h, w, d)


# speedup vs baseline: 2.2548x; 1.1837x over previous
import functools

import jax
import jax.numpy as jnp
from jax import lax
from jax.experimental import pallas as pl
from jax.experimental.pallas import tpu as pltpu
from jax.experimental.pallas import tpu_sc as plsc

_BT = 256
_K = 8192
_D = 256


def _argmin_body(x_ref, e_ref, codes_ref, l2e_ref):
    @pl.when(pl.program_id(0) == 0)
    def _():
        e0 = e_ref[...]
        l2e_ref[...] = jnp.sum(e0 * e0, axis=1)[None, :]

    x = x_ref[...]
    e = e_ref[...]
    dot2 = lax.dot_general(x + x, e, (((1,), (1,)), ((), ())),
                           preferred_element_type=jnp.float32)
    l2x = jnp.sum(x * x, axis=1, keepdims=True)
    l2e = l2e_ref[...]
    dist = (l2x + l2e) - dot2
    codes_ref[0, 0, :] = jnp.argmin(dist, axis=1).astype(jnp.int32)


def _codes_pallas(x, embeddings):
    n = x.shape[0]
    nb = n // _BT
    codes3 = pl.pallas_call(
        _argmin_body,
        grid=(nb,),
        in_specs=[
            pl.BlockSpec((_BT, _D), lambda i: (i, 0)),
            pl.BlockSpec((_K, _D), lambda i: (0, 0)),
        ],
        out_specs=pl.BlockSpec((1, 1, _BT), lambda i: (i, 0, 0)),
        out_shape=jax.ShapeDtypeStruct((nb, 1, _BT), jnp.int32),
        scratch_shapes=[pltpu.VMEM((1, _K), jnp.float32)],
    )(x, embeddings)
    return codes3.reshape(n)


def _gather_sc(embeddings, codes_flat):
    info = plsc.get_sparse_core_info()
    nw = info.num_cores * info.num_subcores
    b = codes_flat.shape[0]
    b_per_w = b // nw
    ch = 256
    nch = b_per_w // ch
    mesh = plsc.VectorSubcoreMesh(core_axis_name="c", subcore_axis_name="s")

    @functools.partial(
        pl.kernel, mesh=mesh,
        out_type=jax.ShapeDtypeStruct((b, _D), jnp.float32),
        scratch_types=[
            pltpu.VMEM((b_per_w,), jnp.int32),
            pltpu.VMEM((ch, _D), jnp.float32),
            pltpu.SemaphoreType.DMA,
        ],
    )
    def k(table_hbm, idx_hbm, out_hbm, idx_v, rows_v, sem):
        wid = lax.axis_index("s") * info.num_cores + lax.axis_index("c")
        base = wid * b_per_w
        pltpu.sync_copy(idx_hbm.at[pl.ds(base, b_per_w)], idx_v)
        for c in range(nch):
            pltpu.async_copy(
                table_hbm.at[idx_v.at[pl.ds(c * ch, ch)]], rows_v, sem
            ).wait()
            pltpu.sync_copy(rows_v, out_hbm.at[pl.ds(base + c * ch, ch)])

    return k(embeddings, codes_flat)


def kernel(inputs, embeddings):
    bsz, h, w, d = inputs.shape
    n = bsz * h * w
    x = inputs.reshape(n, d)
    codes_flat = _codes_pallas(x, embeddings)
    code_vecs = _gather_sc(embeddings, codes_flat)
    return codes_flat.reshape(bsz, h, w), code_vecs.reshape(bsz, h, w, d)


# speedup vs baseline: 2.4049x; 1.0666x over previous
import functools

import jax
import jax.numpy as jnp
from jax import lax
from jax.experimental import pallas as pl
from jax.experimental.pallas import tpu as pltpu
from jax.experimental.pallas import tpu_sc as plsc

_BT = 512
_K = 8192
_D = 256


def _argmin_body(x_ref, e_ref, codes_ref, l2e_ref):
    @pl.when(pl.program_id(0) == 0)
    def _():
        e0 = e_ref[...]
        l2e_ref[...] = jnp.sum(e0 * e0, axis=1)[None, :]

    x = x_ref[...]
    e = e_ref[...]
    dot2 = lax.dot_general(x + x, e, (((1,), (1,)), ((), ())),
                           preferred_element_type=jnp.float32)
    l2x = jnp.sum(x * x, axis=1, keepdims=True)
    l2e = l2e_ref[...]
    dist = (l2x + l2e) - dot2
    codes_ref[0, 0, :] = jnp.argmin(dist, axis=1).astype(jnp.int32)


def _codes_pallas(x, embeddings):
    n = x.shape[0]
    nb = n // _BT
    codes3 = pl.pallas_call(
        _argmin_body,
        grid=(nb,),
        in_specs=[
            pl.BlockSpec((_BT, _D), lambda i: (i, 0)),
            pl.BlockSpec((_K, _D), lambda i: (0, 0)),
        ],
        out_specs=pl.BlockSpec((1, 1, _BT), lambda i: (i, 0, 0)),
        out_shape=jax.ShapeDtypeStruct((nb, 1, _BT), jnp.int32),
        scratch_shapes=[pltpu.VMEM((1, _K), jnp.float32)],
    )(x, embeddings)
    return codes3.reshape(n)


def _gather_sc(embeddings, codes_flat):
    info = plsc.get_sparse_core_info()
    nw = info.num_cores * info.num_subcores
    b = codes_flat.shape[0]
    b_per_w = b // nw
    ch = 256
    nch = b_per_w // ch
    mesh = plsc.VectorSubcoreMesh(core_axis_name="c", subcore_axis_name="s")

    @functools.partial(
        pl.kernel, mesh=mesh,
        out_type=jax.ShapeDtypeStruct((b, _D), jnp.float32),
        scratch_types=[
            pltpu.VMEM((b_per_w,), jnp.int32),
            pltpu.VMEM((ch, _D), jnp.float32),
            pltpu.SemaphoreType.DMA,
        ],
    )
    def k(table_hbm, idx_hbm, out_hbm, idx_v, rows_v, sem):
        wid = lax.axis_index("s") * info.num_cores + lax.axis_index("c")
        base = wid * b_per_w
        pltpu.sync_copy(idx_hbm.at[pl.ds(base, b_per_w)], idx_v)
        for c in range(nch):
            pltpu.async_copy(
                table_hbm.at[idx_v.at[pl.ds(c * ch, ch)]], rows_v, sem
            ).wait()
            pltpu.sync_copy(rows_v, out_hbm.at[pl.ds(base + c * ch, ch)])

    return k(embeddings, codes_flat)


def kernel(inputs, embeddings):
    bsz, h, w, d = inputs.shape
    n = bsz * h * w
    x = inputs.reshape(n, d)
    codes_flat = _codes_pallas(x, embeddings)
    code_vecs = _gather_sc(embeddings, codes_flat)
    return codes_flat.reshape(bsz, h, w), code_vecs.reshape(bsz, h, w, d)
